# baseline (speedup 1.0000x reference)
import functools

import jax
import jax.numpy as jnp
from jax import lax
from jax.experimental import pallas as pl
from jax.experimental.pallas import tpu as pltpu
from jax.experimental.pallas import tpu_sc as plsc

B, D, T = 16, 256, 576
K = 8192
BK = 1024
NKB = K // BK


G = 8
NH = 2
BKH = BK // NH


def _argmin_body(z_ref, emb_ref, idx_ref, minval_ref, zn_ref, zbb_ref,
                 en_ref, ebb_ref, mm2_ref):
    b = pl.program_id(0)
    kblk = pl.program_id(1)

    @pl.when(kblk == 0)
    def _():
        zb = z_ref[0]
        zn_ref[...] = jnp.sum(zb * zb, axis=0, keepdims=True)
        zbb_ref[...] = zb.astype(jnp.bfloat16)

    @pl.when(b == 0)
    def _():
        eb = emb_ref[...]
        en_ref[kblk] = jnp.sum(eb * eb, axis=1, keepdims=True)
        ebb_ref[kblk] = (eb * -2.0).astype(jnp.bfloat16)

    zbb = zbb_ref[...]
    for h in range(NH):
        mm2_ref[h] = lax.dot_general(
            ebb_ref[kblk, pl.ds(h * BKH, BKH), :], zbb,
            (((1,), (0,)), ((), ())),
            preferred_element_type=jnp.float32)

    zn = zn_ref[...]

    def fold(i, carry):
        minv, argi = carry
        ens = en_ref[kblk, pl.ds(i * G, G), :]
        mm2s = mm2_ref[i // (BKH // G), pl.ds((i % (BKH // G)) * G, G), :]
        dd = (zn + ens) + mm2s
        win = dd < minv
        argi = jnp.where(win, i.astype(jnp.float32), argi)
        minv = jnp.where(win, dd, minv)
        return minv, argi

    minv0 = jnp.full((G, T), jnp.inf, dtype=jnp.float32)
    argi0 = jnp.zeros((G, T), dtype=jnp.float32)
    minv, argi = lax.fori_loop(0, BK // G, fold, (minv0, argi0), unroll=8)

    srow = lax.broadcasted_iota(jnp.int32, (G, T), 0).astype(jnp.float32)
    vidx = argi * jnp.float32(G) + srow
    h = G
    while h > 1:
        h //= 2
        va, vb = minv[:h], minv[h:]
        ia, ib = vidx[:h], vidx[h:]
        ta = (va < vb) | ((va == vb) & (ia < ib))
        minv = jnp.where(ta, va, vb)
        vidx = jnp.where(ta, ia, ib)
    lmin = minv
    larg = vidx.astype(jnp.int32) + kblk * BK

    @pl.when(kblk == 0)
    def _():
        minval_ref[...] = lmin
        idx_ref[0] = larg

    @pl.when(kblk > 0)
    def _():
        prev = minval_ref[...]
        better = lmin < prev
        minval_ref[...] = jnp.where(better, lmin, prev)
        idx_ref[0] = jnp.where(better, larg, idx_ref[0])


def _encode_indices(z, embedding):
    return pl.pallas_call(
        _argmin_body,
        grid=(B, NKB),
        in_specs=[
            pl.BlockSpec((1, D, T), lambda b, k: (b, 0, 0)),
            pl.BlockSpec((BK, D), lambda b, k: (k, 0)),
        ],
        out_specs=pl.BlockSpec((1, 1, T), lambda b, k: (b, 0, 0)),
        out_shape=jax.ShapeDtypeStruct((B, 1, T), jnp.int32),
        scratch_shapes=[
            pltpu.VMEM((1, T), jnp.float32),
            pltpu.VMEM((1, T), jnp.float32),
            pltpu.VMEM((D, T), jnp.bfloat16),
            pltpu.VMEM((NKB, BK, 1), jnp.float32),
            pltpu.VMEM((NKB, BK, D), jnp.bfloat16),
            pltpu.VMEM((NH, BKH, T), jnp.float32),
        ],
    )(z, embedding)


def _transpose_body(in_ref, out_ref):
    out_ref[0] = in_ref[0].T


def _transpose_btd(x):
    return pl.pallas_call(
        _transpose_body,
        grid=(B,),
        in_specs=[pl.BlockSpec((1, T, D), lambda b: (b, 0, 0))],
        out_specs=pl.BlockSpec((1, D, T), lambda b: (b, 0, 0)),
        out_shape=jax.ShapeDtypeStruct((B, D, T), jnp.float32),
    )(x)


def _make_sc_gather():
    info = plsc.get_sparse_core_info()
    nw = info.num_cores * info.num_subcores
    rows = B * T
    b_per_w = rows // nw
    mesh = plsc.VectorSubcoreMesh(core_axis_name="c", subcore_axis_name="s")

    @functools.partial(
        pl.kernel, mesh=mesh,
        out_type=jax.ShapeDtypeStruct((rows, D), jnp.float32),
        scratch_types=[
            pltpu.VMEM((b_per_w,), jnp.int32),
            pltpu.VMEM((b_per_w, D), jnp.float32),
            pltpu.SemaphoreType.DMA,
        ],
    )
    def gather_rows(table_hbm, idx_hbm, out_hbm, idx_v, rows_v, sem):
        wid = lax.axis_index("s") * info.num_cores + lax.axis_index("c")
        base = wid * b_per_w
        pltpu.sync_copy(idx_hbm.at[pl.ds(base, b_per_w)], idx_v)
        pltpu.async_copy(table_hbm.at[idx_v], rows_v, sem).wait()
        pltpu.sync_copy(rows_v, out_hbm.at[pl.ds(base, b_per_w)])

    return gather_rows


def kernel(z, embedding):
    idx = _encode_indices(z, embedding)
    idx_flat = idx.reshape(B * T)
    gathered = _make_sc_gather()(embedding, idx_flat)
    return _transpose_btd(gathered.reshape(B, T, D))


# speedup vs baseline: 1.5579x; 1.5579x over previous
import functools

import jax
import jax.numpy as jnp
from jax import lax
from jax.experimental import pallas as pl
from jax.experimental.pallas import tpu as pltpu
from jax.experimental.pallas import tpu_sc as plsc

B, D, T = 16, 256, 576
K = 8192
BK = 1024
NKB = K // BK


G = 8
NH = 8
BKH = BK // NH
J = 8


def _argmin_body(z_ref, emb_ref, idx_ref, minval_ref, zn_ref, zbb_ref,
                 en_ref, ebb_ref, mm2_ref):
    b = pl.program_id(0)
    kblk = pl.program_id(1)

    @pl.when(kblk == 0)
    def _():
        zb = z_ref[0]
        zn_ref[...] = jnp.sum(zb * zb, axis=0, keepdims=True)
        zbb_ref[...] = zb.astype(jnp.bfloat16)

    @pl.when(b == 0)
    def _():
        eb = emb_ref[...]
        en_ref[kblk] = jnp.sum(eb * eb, axis=1, keepdims=True)
        ebb_ref[kblk] = (eb * -2.0).astype(jnp.bfloat16)

    zbb = zbb_ref[...]
    for h in range(NH):
        mm2_ref[h] = lax.dot_general(
            ebb_ref[kblk, pl.ds(h * BKH, BKH), :], zbb,
            (((1,), (0,)), ((), ())),
            preferred_element_type=jnp.float32)

    zn = zn_ref[...]

    gph = BKH // G
    minvs = [jnp.full((G, T), jnp.inf, dtype=jnp.float32) for _ in range(J)]
    argis = [jnp.zeros((G, T), dtype=jnp.float32) for _ in range(J)]
    for g in range(BK // G):
        j = g % J
        ens = en_ref[kblk, pl.ds(g * G, G), :]
        mm2s = mm2_ref[g // gph, pl.ds((g % gph) * G, G), :]
        dd = (zn + ens) + mm2s
        win = dd < minvs[j]
        argis[j] = jnp.where(win, jnp.float32(g), argis[j])
        minvs[j] = jnp.where(win, dd, minvs[j])

    srow = lax.broadcasted_iota(jnp.int32, (G, T), 0).astype(jnp.float32)
    vidxs = [a * jnp.float32(G) + srow for a in argis]
    nacc = J
    while nacc > 1:
        nacc //= 2
        for j in range(nacc):
            va, vb = minvs[j], minvs[j + nacc]
            ia, ib = vidxs[j], vidxs[j + nacc]
            ta = (va < vb) | ((va == vb) & (ia < ib))
            minvs[j] = jnp.where(ta, va, vb)
            vidxs[j] = jnp.where(ta, ia, ib)
    minv, vidx = minvs[0], vidxs[0]
    h = G
    while h > 1:
        h //= 2
        va, vb = minv[:h], minv[h:]
        ia, ib = vidx[:h], vidx[h:]
        ta = (va < vb) | ((va == vb) & (ia < ib))
        minv = jnp.where(ta, va, vb)
        vidx = jnp.where(ta, ia, ib)
    lmin = minv
    larg = vidx.astype(jnp.int32) + kblk * BK

    @pl.when(kblk == 0)
    def _():
        minval_ref[...] = lmin
        idx_ref[0] = larg

    @pl.when(kblk > 0)
    def _():
        prev = minval_ref[...]
        better = lmin < prev
        minval_ref[...] = jnp.where(better, lmin, prev)
        idx_ref[0] = jnp.where(better, larg, idx_ref[0])


def _encode_indices(z, embedding):
    return pl.pallas_call(
        _argmin_body,
        grid=(B, NKB),
        in_specs=[
            pl.BlockSpec((1, D, T), lambda b, k: (b, 0, 0)),
            pl.BlockSpec((BK, D), lambda b, k: (k, 0)),
        ],
        out_specs=pl.BlockSpec((1, 1, T), lambda b, k: (b, 0, 0)),
        out_shape=jax.ShapeDtypeStruct((B, 1, T), jnp.int32),
        scratch_shapes=[
            pltpu.VMEM((1, T), jnp.float32),
            pltpu.VMEM((1, T), jnp.float32),
            pltpu.VMEM((D, T), jnp.bfloat16),
            pltpu.VMEM((NKB, BK, 1), jnp.float32),
            pltpu.VMEM((NKB, BK, D), jnp.bfloat16),
            pltpu.VMEM((NH, BKH, T), jnp.float32),
        ],
    )(z, embedding)


def _transpose_body(in_ref, out_ref):
    out_ref[0] = in_ref[0].T


def _transpose_btd(x):
    return pl.pallas_call(
        _transpose_body,
        grid=(B,),
        in_specs=[pl.BlockSpec((1, T, D), lambda b: (b, 0, 0))],
        out_specs=pl.BlockSpec((1, D, T), lambda b: (b, 0, 0)),
        out_shape=jax.ShapeDtypeStruct((B, D, T), jnp.float32),
    )(x)


def _make_sc_gather():
    info = plsc.get_sparse_core_info()
    nw = info.num_cores * info.num_subcores
    rows = B * T
    b_per_w = rows // nw
    mesh = plsc.VectorSubcoreMesh(core_axis_name="c", subcore_axis_name="s")

    @functools.partial(
        pl.kernel, mesh=mesh,
        out_type=jax.ShapeDtypeStruct((rows, D), jnp.float32),
        scratch_types=[
            pltpu.VMEM((b_per_w,), jnp.int32),
            pltpu.VMEM((b_per_w, D), jnp.float32),
            pltpu.SemaphoreType.DMA,
        ],
    )
    def gather_rows(table_hbm, idx_hbm, out_hbm, idx_v, rows_v, sem):
        wid = lax.axis_index("s") * info.num_cores + lax.axis_index("c")
        base = wid * b_per_w
        pltpu.sync_copy(idx_hbm.at[pl.ds(base, b_per_w)], idx_v)
        pltpu.async_copy(table_hbm.at[idx_v], rows_v, sem).wait()
        pltpu.sync_copy(rows_v, out_hbm.at[pl.ds(base, b_per_w)])

    return gather_rows


def kernel(z, embedding):
    idx = _encode_indices(z, embedding)
    idx_flat = idx.reshape(B * T)
    gathered = _make_sc_gather()(embedding, idx_flat)
    return _transpose_btd(gathered.reshape(B, T, D))


# speedup vs baseline: 1.7303x; 1.1106x over previous
import functools

import jax
import jax.numpy as jnp
from jax import lax
from jax.experimental import pallas as pl
from jax.experimental.pallas import tpu as pltpu
from jax.experimental.pallas import tpu_sc as plsc

B, D, T = 16, 256, 576
K = 8192
BK = 2048
NKB = K // BK
G = 8
NH = 8
BKH = BK // NH
J = 8


def _emb_prep_body(emb_ref, ebb_ref, en_ref):
    eb = emb_ref[...]
    ebb_ref[0] = (eb * -2.0).astype(jnp.bfloat16)
    en_ref[0] = jnp.sum(eb * eb, axis=1, keepdims=True)


def _emb_prep(embedding):
    return pl.pallas_call(
        _emb_prep_body,
        grid=(NKB,),
        in_specs=[pl.BlockSpec((BK, D), lambda k: (k, 0))],
        out_specs=[
            pl.BlockSpec((1, BK, D), lambda k: (k, 0, 0)),
            pl.BlockSpec((1, BK, 1), lambda k: (k, 0, 0)),
        ],
        out_shape=[
            jax.ShapeDtypeStruct((NKB, BK, D), jnp.bfloat16),
            jax.ShapeDtypeStruct((NKB, BK, 1), jnp.float32),
        ],
    )(embedding)


def _z_prep_body(z_ref, zbb_ref, zn_ref):
    zb = z_ref[0]
    zbb_ref[0] = zb.astype(jnp.bfloat16)
    zn_ref[0] = jnp.sum(zb * zb, axis=0, keepdims=True)


def _z_prep(z):
    return pl.pallas_call(
        _z_prep_body,
        grid=(B,),
        in_specs=[pl.BlockSpec((1, D, T), lambda b: (b, 0, 0))],
        out_specs=[
            pl.BlockSpec((1, D, T), lambda b: (b, 0, 0)),
            pl.BlockSpec((1, 1, T), lambda b: (b, 0, 0)),
        ],
        out_shape=[
            jax.ShapeDtypeStruct((B, D, T), jnp.bfloat16),
            jax.ShapeDtypeStruct((B, 1, T), jnp.float32),
        ],
    )(z)


def _argmin_body(zbb_ref, zn_ref, ebb_ref, en_ref, idx_ref, minval_ref,
                 mm2_ref):
    kblk = pl.program_id(1)

    zbb = zbb_ref[0]
    for h in range(NH):
        mm2_ref[h] = lax.dot_general(
            ebb_ref[0, pl.ds(h * BKH, BKH), :], zbb,
            (((1,), (0,)), ((), ())),
            preferred_element_type=jnp.float32)

    zn = zn_ref[0]

    gph = BKH // G
    minvs = [jnp.full((G, T), jnp.inf, dtype=jnp.float32) for _ in range(J)]
    argis = [jnp.zeros((G, T), dtype=jnp.float32) for _ in range(J)]
    for g in range(BK // G):
        j = g % J
        ens = en_ref[0, pl.ds(g * G, G), :]
        mm2s = mm2_ref[g // gph, pl.ds((g % gph) * G, G), :]
        dd = (zn + ens) + mm2s
        win = dd < minvs[j]
        argis[j] = jnp.where(win, jnp.float32(g), argis[j])
        minvs[j] = jnp.where(win, dd, minvs[j])

    srow = lax.broadcasted_iota(jnp.int32, (G, T), 0).astype(jnp.float32)
    vidxs = [a * jnp.float32(G) + srow for a in argis]
    nacc = J
    while nacc > 1:
        nacc //= 2
        for j in range(nacc):
            va, vb = minvs[j], minvs[j + nacc]
            ia, ib = vidxs[j], vidxs[j + nacc]
            ta = (va < vb) | ((va == vb) & (ia < ib))
            minvs[j] = jnp.where(ta, va, vb)
            vidxs[j] = jnp.where(ta, ia, ib)
    minv, vidx = minvs[0], vidxs[0]
    h = G
    while h > 1:
        h //= 2
        va, vb = minv[:h], minv[h:]
        ia, ib = vidx[:h], vidx[h:]
        ta = (va < vb) | ((va == vb) & (ia < ib))
        minv = jnp.where(ta, va, vb)
        vidx = jnp.where(ta, ia, ib)
    lmin = minv
    larg = vidx.astype(jnp.int32) + kblk * BK

    prev = minval_ref[...]
    better = jnp.logical_or(lmin < prev, kblk == 0)
    minval_ref[...] = jnp.where(better, lmin, prev)
    idx_ref[0] = jnp.where(better, larg, idx_ref[0])


def _encode_indices(zbb, zn, ebb, en):
    return pl.pallas_call(
        _argmin_body,
        grid=(B, NKB),
        in_specs=[
            pl.BlockSpec((1, D, T), lambda b, k: (b, 0, 0)),
            pl.BlockSpec((1, 1, T), lambda b, k: (b, 0, 0)),
            pl.BlockSpec((1, BK, D), lambda b, k: (k, 0, 0)),
            pl.BlockSpec((1, BK, 1), lambda b, k: (k, 0, 0)),
        ],
        out_specs=pl.BlockSpec((1, 1, T), lambda b, k: (b, 0, 0)),
        out_shape=jax.ShapeDtypeStruct((B, 1, T), jnp.int32),
        scratch_shapes=[
            pltpu.VMEM((1, T), jnp.float32),
            pltpu.VMEM((NH, BKH, T), jnp.float32),
        ],
    )(zbb, zn, ebb, en)


def _transpose_body(in_ref, out_ref):
    out_ref[0] = in_ref[0].T


def _transpose_btd(x):
    return pl.pallas_call(
        _transpose_body,
        grid=(B,),
        in_specs=[pl.BlockSpec((1, T, D), lambda b: (b, 0, 0))],
        out_specs=pl.BlockSpec((1, D, T), lambda b: (b, 0, 0)),
        out_shape=jax.ShapeDtypeStruct((B, D, T), jnp.float32),
    )(x)


def _make_sc_gather():
    info = plsc.get_sparse_core_info()
    nw = info.num_cores * info.num_subcores
    rows = B * T
    b_per_w = rows // nw
    mesh = plsc.VectorSubcoreMesh(core_axis_name="c", subcore_axis_name="s")

    @functools.partial(
        pl.kernel, mesh=mesh,
        out_type=jax.ShapeDtypeStruct((rows, D), jnp.float32),
        scratch_types=[
            pltpu.VMEM((b_per_w,), jnp.int32),
            pltpu.VMEM((b_per_w, D), jnp.float32),
            pltpu.SemaphoreType.DMA,
        ],
    )
    def gather_rows(table_hbm, idx_hbm, out_hbm, idx_v, rows_v, sem):
        wid = lax.axis_index("s") * info.num_cores + lax.axis_index("c")
        base = wid * b_per_w
        pltpu.sync_copy(idx_hbm.at[pl.ds(base, b_per_w)], idx_v)
        pltpu.async_copy(table_hbm.at[idx_v], rows_v, sem).wait()
        pltpu.sync_copy(rows_v, out_hbm.at[pl.ds(base, b_per_w)])

    return gather_rows


def kernel(z, embedding):
    ebb, en = _emb_prep(embedding)
    zbb, zn = _z_prep(z)
    idx = _encode_indices(zbb, zn, ebb, en)
    idx_flat = idx.reshape(B * T)
    gathered = _make_sc_gather()(embedding, idx_flat)
    return _transpose_btd(gathered.reshape(B, T, D))


# speedup vs baseline: 1.9053x; 1.1011x over previous
import functools

import jax
import jax.numpy as jnp
from jax import lax
from jax.experimental import pallas as pl
from jax.experimental.pallas import tpu as pltpu
from jax.experimental.pallas import tpu_sc as plsc

B, D, T = 16, 256, 576
K = 8192
BK = 2048
NKB = K // BK
G = 8
NH = 8
BKH = BK // NH
J = 8


EPB = K // B


def _prep_body(z_ref, emb_ref, zbb_ref, zn_ref, ebb_ref, en_ref):
    zb = z_ref[0]
    zbb_ref[0] = zb.astype(jnp.bfloat16)
    zn_ref[0] = jnp.sum(zb * zb, axis=0, keepdims=True)
    eb = emb_ref[...]
    ebb_ref[0] = (eb * -2.0).astype(jnp.bfloat16)
    en_ref[0] = jnp.sum(eb * eb, axis=1, keepdims=True)


def _prep(z, embedding):
    zbb, zn, ebb, en = pl.pallas_call(
        _prep_body,
        grid=(B,),
        in_specs=[
            pl.BlockSpec((1, D, T), lambda b: (b, 0, 0)),
            pl.BlockSpec((EPB, D), lambda b: (b, 0)),
        ],
        out_specs=[
            pl.BlockSpec((1, D, T), lambda b: (b, 0, 0)),
            pl.BlockSpec((1, 1, T), lambda b: (b, 0, 0)),
            pl.BlockSpec((1, EPB, D), lambda b: (b, 0, 0)),
            pl.BlockSpec((1, EPB, 1), lambda b: (b, 0, 0)),
        ],
        out_shape=[
            jax.ShapeDtypeStruct((B, D, T), jnp.bfloat16),
            jax.ShapeDtypeStruct((B, 1, T), jnp.float32),
            jax.ShapeDtypeStruct((B, EPB, D), jnp.bfloat16),
            jax.ShapeDtypeStruct((B, EPB, 1), jnp.float32),
        ],
    )(z, embedding)
    return (zbb, zn, ebb.reshape(NKB, BK, D), en.reshape(NKB, BK, 1))


def _argmin_body(zbb_ref, zn_ref, ebb_ref, en_ref, idx_ref, minval_ref):
    kblk = pl.program_id(1)
    zn = zn_ref[0]

    gph = BKH // G
    minvs = [jnp.full((G, T), jnp.inf, dtype=jnp.float32) for _ in range(J)]
    argis = [jnp.zeros((G, T), dtype=jnp.float32) for _ in range(J)]
    for h in range(NH):
        mm2h = lax.dot_general(
            ebb_ref[0, pl.ds(h * BKH, BKH), :], zbb_ref[0],
            (((1,), (0,)), ((), ())),
            preferred_element_type=jnp.float32)
        for gg in range(gph):
            g = h * gph + gg
            j = g % J
            ens = en_ref[0, pl.ds(g * G, G), :]
            mm2s = mm2h[gg * G:(gg + 1) * G, :]
            dd = (zn + ens) + mm2s
            win = dd < minvs[j]
            argis[j] = jnp.where(win, jnp.float32(g), argis[j])
            minvs[j] = jnp.where(win, dd, minvs[j])

    srow = lax.broadcasted_iota(jnp.int32, (G, T), 0).astype(jnp.float32)
    vidxs = [a * jnp.float32(G) + srow for a in argis]
    nacc = J
    while nacc > 1:
        nacc //= 2
        for j in range(nacc):
            va, vb = minvs[j], minvs[j + nacc]
            ia, ib = vidxs[j], vidxs[j + nacc]
            ta = (va < vb) | ((va == vb) & (ia < ib))
            minvs[j] = jnp.where(ta, va, vb)
            vidxs[j] = jnp.where(ta, ia, ib)
    minv, vidx = minvs[0], vidxs[0]
    h = G
    while h > 1:
        h //= 2
        va, vb = minv[:h], minv[h:]
        ia, ib = vidx[:h], vidx[h:]
        ta = (va < vb) | ((va == vb) & (ia < ib))
        minv = jnp.where(ta, va, vb)
        vidx = jnp.where(ta, ia, ib)
    lmin = minv
    larg = vidx.astype(jnp.int32) + kblk * BK

    prev = minval_ref[...]
    better = jnp.logical_or(lmin < prev, kblk == 0)
    minval_ref[...] = jnp.where(better, lmin, prev)
    idx_ref[0] = jnp.where(better, larg, idx_ref[0])


def _encode_indices(zbb, zn, ebb, en):
    return pl.pallas_call(
        _argmin_body,
        grid=(B, NKB),
        in_specs=[
            pl.BlockSpec((1, D, T), lambda b, k: (b, 0, 0)),
            pl.BlockSpec((1, 1, T), lambda b, k: (b, 0, 0)),
            pl.BlockSpec((1, BK, D), lambda b, k: (k, 0, 0)),
            pl.BlockSpec((1, BK, 1), lambda b, k: (k, 0, 0)),
        ],
        out_specs=pl.BlockSpec((1, 1, T), lambda b, k: (b, 0, 0)),
        out_shape=jax.ShapeDtypeStruct((B, 1, T), jnp.int32),
        scratch_shapes=[
            pltpu.VMEM((1, T), jnp.float32),
        ],
    )(zbb, zn, ebb, en)


def _transpose_body(in_ref, out_ref):
    out_ref[0] = in_ref[0].T


def _transpose_btd(x):
    return pl.pallas_call(
        _transpose_body,
        grid=(B,),
        in_specs=[pl.BlockSpec((1, T, D), lambda b: (b, 0, 0))],
        out_specs=pl.BlockSpec((1, D, T), lambda b: (b, 0, 0)),
        out_shape=jax.ShapeDtypeStruct((B, D, T), jnp.float32),
    )(x)


def _make_sc_gather():
    info = plsc.get_sparse_core_info()
    nw = info.num_cores * info.num_subcores
    rows = B * T
    b_per_w = rows // nw
    mesh = plsc.VectorSubcoreMesh(core_axis_name="c", subcore_axis_name="s")

    @functools.partial(
        pl.kernel, mesh=mesh,
        out_type=jax.ShapeDtypeStruct((rows, D), jnp.float32),
        scratch_types=[
            pltpu.VMEM((b_per_w,), jnp.int32),
            pltpu.VMEM((b_per_w, D), jnp.float32),
            pltpu.SemaphoreType.DMA,
        ],
    )
    def gather_rows(table_hbm, idx_hbm, out_hbm, idx_v, rows_v, sem):
        wid = lax.axis_index("s") * info.num_cores + lax.axis_index("c")
        base = wid * b_per_w
        pltpu.sync_copy(idx_hbm.at[pl.ds(base, b_per_w)], idx_v)
        pltpu.async_copy(table_hbm.at[idx_v], rows_v, sem).wait()
        pltpu.sync_copy(rows_v, out_hbm.at[pl.ds(base, b_per_w)])

    return gather_rows


def kernel(z, embedding):
    zbb, zn, ebb, en = _prep(z, embedding)
    idx = _encode_indices(zbb, zn, ebb, en)
    idx_flat = idx.reshape(B * T)
    gathered = _make_sc_gather()(embedding, idx_flat)
    return _transpose_btd(gathered.reshape(B, T, D))


# speedup vs baseline: 2.0316x; 1.0663x over previous
import functools

import jax
import jax.numpy as jnp
from jax import lax
from jax.experimental import pallas as pl
from jax.experimental.pallas import tpu as pltpu
from jax.experimental.pallas import tpu_sc as plsc

B, D, T = 16, 256, 576
K = 8192
BK = 4096
NKB = K // BK
G = 8
NH = 16
BKH = BK // NH
J = 8


EPB = K // B


def _prep_body(z_ref, emb_ref, zbb_ref, zn_ref, ebb_ref, en_ref):
    zb = z_ref[0]
    zbb_ref[0] = zb.astype(jnp.bfloat16)
    zn_ref[0] = jnp.sum(zb * zb, axis=0, keepdims=True)
    eb = emb_ref[...]
    ebb_ref[0] = (eb * -2.0).astype(jnp.bfloat16)
    en_ref[0] = jnp.sum(eb * eb, axis=1, keepdims=True)


def _prep(z, embedding):
    zbb, zn, ebb, en = pl.pallas_call(
        _prep_body,
        grid=(B,),
        in_specs=[
            pl.BlockSpec((1, D, T), lambda b: (b, 0, 0)),
            pl.BlockSpec((EPB, D), lambda b: (b, 0)),
        ],
        out_specs=[
            pl.BlockSpec((1, D, T), lambda b: (b, 0, 0)),
            pl.BlockSpec((1, 1, T), lambda b: (b, 0, 0)),
            pl.BlockSpec((1, EPB, D), lambda b: (b, 0, 0)),
            pl.BlockSpec((1, EPB, 1), lambda b: (b, 0, 0)),
        ],
        out_shape=[
            jax.ShapeDtypeStruct((B, D, T), jnp.bfloat16),
            jax.ShapeDtypeStruct((B, 1, T), jnp.float32),
            jax.ShapeDtypeStruct((B, EPB, D), jnp.bfloat16),
            jax.ShapeDtypeStruct((B, EPB, 1), jnp.float32),
        ],
    )(z, embedding)
    return (zbb, zn, ebb.reshape(NKB, BK, D), en.reshape(NKB, BK, 1))


def _argmin_body(zbb_ref, zn_ref, ebb_ref, en_ref, idx_ref, minval_ref):
    kblk = pl.program_id(1)
    zn = zn_ref[0]

    gph = BKH // G
    minvs = [jnp.full((G, T), jnp.inf, dtype=jnp.float32) for _ in range(J)]
    argis = [jnp.zeros((G, T), dtype=jnp.float32) for _ in range(J)]
    for h in range(NH):
        mm2h = lax.dot_general(
            ebb_ref[0, pl.ds(h * BKH, BKH), :], zbb_ref[0],
            (((1,), (0,)), ((), ())),
            preferred_element_type=jnp.float32)
        for gg in range(gph):
            g = h * gph + gg
            j = g % J
            ens = en_ref[0, pl.ds(g * G, G), :]
            mm2s = mm2h[gg * G:(gg + 1) * G, :]
            dd = (zn + ens) + mm2s
            win = dd < minvs[j]
            argis[j] = jnp.where(win, jnp.float32(g), argis[j])
            minvs[j] = jnp.where(win, dd, minvs[j])

    srow = lax.broadcasted_iota(jnp.int32, (G, T), 0).astype(jnp.float32)
    vidxs = [a * jnp.float32(G) + srow for a in argis]
    nacc = J
    while nacc > 1:
        nacc //= 2
        for j in range(nacc):
            va, vb = minvs[j], minvs[j + nacc]
            ia, ib = vidxs[j], vidxs[j + nacc]
            ta = (va < vb) | ((va == vb) & (ia < ib))
            minvs[j] = jnp.where(ta, va, vb)
            vidxs[j] = jnp.where(ta, ia, ib)
    minv, vidx = minvs[0], vidxs[0]
    h = G
    while h > 1:
        h //= 2
        va, vb = minv[:h], minv[h:]
        ia, ib = vidx[:h], vidx[h:]
        ta = (va < vb) | ((va == vb) & (ia < ib))
        minv = jnp.where(ta, va, vb)
        vidx = jnp.where(ta, ia, ib)
    lmin = minv
    larg = vidx.astype(jnp.int32) + kblk * BK

    prev = minval_ref[...]
    better = jnp.logical_or(lmin < prev, kblk == 0)
    minval_ref[...] = jnp.where(better, lmin, prev)
    idx_ref[0] = jnp.where(better, larg, idx_ref[0])


def _encode_indices(zbb, zn, ebb, en):
    return pl.pallas_call(
        _argmin_body,
        grid=(B, NKB),
        in_specs=[
            pl.BlockSpec((1, D, T), lambda b, k: (b, 0, 0)),
            pl.BlockSpec((1, 1, T), lambda b, k: (b, 0, 0)),
            pl.BlockSpec((1, BK, D), lambda b, k: (k, 0, 0)),
            pl.BlockSpec((1, BK, 1), lambda b, k: (k, 0, 0)),
        ],
        out_specs=pl.BlockSpec((1, 1, T), lambda b, k: (b, 0, 0)),
        out_shape=jax.ShapeDtypeStruct((B, 1, T), jnp.int32),
        scratch_shapes=[
            pltpu.VMEM((1, T), jnp.float32),
        ],
    )(zbb, zn, ebb, en)


TB = 2


def _transpose_body(in_ref, out_ref):
    for i in range(TB):
        out_ref[i] = in_ref[i].T


def _transpose_btd(x):
    return pl.pallas_call(
        _transpose_body,
        grid=(B // TB,),
        in_specs=[pl.BlockSpec((TB, T, D), lambda b: (b, 0, 0))],
        out_specs=pl.BlockSpec((TB, D, T), lambda b: (b, 0, 0)),
        out_shape=jax.ShapeDtypeStruct((B, D, T), jnp.float32),
    )(x)


def _make_sc_gather():
    info = plsc.get_sparse_core_info()
    nw = info.num_cores * info.num_subcores
    rows = B * T
    b_per_w = rows // nw
    mesh = plsc.VectorSubcoreMesh(core_axis_name="c", subcore_axis_name="s")

    @functools.partial(
        pl.kernel, mesh=mesh,
        out_type=jax.ShapeDtypeStruct((rows, D), jnp.float32),
        scratch_types=[
            pltpu.VMEM((b_per_w,), jnp.int32),
            pltpu.VMEM((b_per_w, D), jnp.float32),
            pltpu.SemaphoreType.DMA,
        ],
    )
    def gather_rows(table_hbm, idx_hbm, out_hbm, idx_v, rows_v, sem):
        wid = lax.axis_index("s") * info.num_cores + lax.axis_index("c")
        base = wid * b_per_w
        pltpu.sync_copy(idx_hbm.at[pl.ds(base, b_per_w)], idx_v)
        pltpu.async_copy(table_hbm.at[idx_v], rows_v, sem).wait()
        pltpu.sync_copy(rows_v, out_hbm.at[pl.ds(base, b_per_w)])

    return gather_rows


def kernel(z, embedding):
    zbb, zn, ebb, en = _prep(z, embedding)
    idx = _encode_indices(zbb, zn, ebb, en)
    idx_flat = idx.reshape(B * T)
    gathered = _make_sc_gather()(embedding, idx_flat)
    return _transpose_btd(gathered.reshape(B, T, D))


# speedup vs baseline: 2.0369x; 1.0026x over previous
import functools

import jax
import jax.numpy as jnp
from jax import lax
from jax.experimental import pallas as pl
from jax.experimental.pallas import tpu as pltpu
from jax.experimental.pallas import tpu_sc as plsc

B, D, T = 16, 256, 576
K = 8192
BK = 8192
NKB = K // BK
G = 8
NH = 32
BKH = BK // NH
J = 8


EPB = K // B


def _prep_body(z_ref, emb_ref, zbb_ref, zn_ref, ebb_ref, en_ref):
    zb = z_ref[0]
    zbb_ref[0] = zb.astype(jnp.bfloat16)
    zn_ref[0] = jnp.sum(zb * zb, axis=0, keepdims=True)
    eb = emb_ref[...]
    ebb_ref[0] = (eb * -2.0).astype(jnp.bfloat16)
    en_ref[0] = jnp.sum(eb * eb, axis=1, keepdims=True)


def _prep(z, embedding):
    zbb, zn, ebb, en = pl.pallas_call(
        _prep_body,
        grid=(B,),
        in_specs=[
            pl.BlockSpec((1, D, T), lambda b: (b, 0, 0)),
            pl.BlockSpec((EPB, D), lambda b: (b, 0)),
        ],
        out_specs=[
            pl.BlockSpec((1, D, T), lambda b: (b, 0, 0)),
            pl.BlockSpec((1, 1, T), lambda b: (b, 0, 0)),
            pl.BlockSpec((1, EPB, D), lambda b: (b, 0, 0)),
            pl.BlockSpec((1, EPB, 1), lambda b: (b, 0, 0)),
        ],
        out_shape=[
            jax.ShapeDtypeStruct((B, D, T), jnp.bfloat16),
            jax.ShapeDtypeStruct((B, 1, T), jnp.float32),
            jax.ShapeDtypeStruct((B, EPB, D), jnp.bfloat16),
            jax.ShapeDtypeStruct((B, EPB, 1), jnp.float32),
        ],
    )(z, embedding)
    return (zbb, zn, ebb.reshape(NKB, BK, D), en.reshape(NKB, BK, 1))


def _argmin_body(zbb_ref, zn_ref, ebb_ref, en_ref, idx_ref, minval_ref):
    kblk = pl.program_id(1)
    zn = zn_ref[0]

    gph = BKH // G
    minvs = [jnp.full((G, T), jnp.inf, dtype=jnp.float32) for _ in range(J)]
    argis = [jnp.zeros((G, T), dtype=jnp.float32) for _ in range(J)]
    for h in range(NH):
        mm2h = lax.dot_general(
            ebb_ref[0, pl.ds(h * BKH, BKH), :], zbb_ref[0],
            (((1,), (0,)), ((), ())),
            preferred_element_type=jnp.float32)
        for gg in range(gph):
            g = h * gph + gg
            j = g % J
            ens = en_ref[0, pl.ds(g * G, G), :]
            mm2s = mm2h[gg * G:(gg + 1) * G, :]
            dd = (zn + ens) + mm2s
            win = dd < minvs[j]
            argis[j] = jnp.where(win, jnp.float32(g), argis[j])
            minvs[j] = jnp.where(win, dd, minvs[j])

    srow = lax.broadcasted_iota(jnp.int32, (G, T), 0).astype(jnp.float32)
    vidxs = [a * jnp.float32(G) + srow for a in argis]
    nacc = J
    while nacc > 1:
        nacc //= 2
        for j in range(nacc):
            va, vb = minvs[j], minvs[j + nacc]
            ia, ib = vidxs[j], vidxs[j + nacc]
            ta = (va < vb) | ((va == vb) & (ia < ib))
            minvs[j] = jnp.where(ta, va, vb)
            vidxs[j] = jnp.where(ta, ia, ib)
    minv, vidx = minvs[0], vidxs[0]
    h = G
    while h > 1:
        h //= 2
        va, vb = minv[:h], minv[h:]
        ia, ib = vidx[:h], vidx[h:]
        ta = (va < vb) | ((va == vb) & (ia < ib))
        minv = jnp.where(ta, va, vb)
        vidx = jnp.where(ta, ia, ib)
    lmin = minv
    larg = vidx.astype(jnp.int32) + kblk * BK

    prev = minval_ref[...]
    better = jnp.logical_or(lmin < prev, kblk == 0)
    minval_ref[...] = jnp.where(better, lmin, prev)
    idx_ref[0] = jnp.where(better, larg, idx_ref[0])


def _encode_indices(zbb, zn, ebb, en):
    return pl.pallas_call(
        _argmin_body,
        grid=(B, NKB),
        in_specs=[
            pl.BlockSpec((1, D, T), lambda b, k: (b, 0, 0)),
            pl.BlockSpec((1, 1, T), lambda b, k: (b, 0, 0)),
            pl.BlockSpec((1, BK, D), lambda b, k: (k, 0, 0)),
            pl.BlockSpec((1, BK, 1), lambda b, k: (k, 0, 0)),
        ],
        out_specs=pl.BlockSpec((1, 1, T), lambda b, k: (b, 0, 0)),
        out_shape=jax.ShapeDtypeStruct((B, 1, T), jnp.int32),
        scratch_shapes=[
            pltpu.VMEM((1, T), jnp.float32),
        ],
    )(zbb, zn, ebb, en)


TB = 2


def _transpose_body(in_ref, out_ref):
    for i in range(TB):
        out_ref[i] = in_ref[i].T


def _transpose_btd(x):
    return pl.pallas_call(
        _transpose_body,
        grid=(B // TB,),
        in_specs=[pl.BlockSpec((TB, T, D), lambda b: (b, 0, 0))],
        out_specs=pl.BlockSpec((TB, D, T), lambda b: (b, 0, 0)),
        out_shape=jax.ShapeDtypeStruct((B, D, T), jnp.float32),
    )(x)


def _make_sc_gather():
    info = plsc.get_sparse_core_info()
    nw = info.num_cores * info.num_subcores
    rows = B * T
    b_per_w = rows // nw
    mesh = plsc.VectorSubcoreMesh(core_axis_name="c", subcore_axis_name="s")

    @functools.partial(
        pl.kernel, mesh=mesh,
        out_type=jax.ShapeDtypeStruct((rows, D), jnp.float32),
        scratch_types=[
            pltpu.VMEM((b_per_w,), jnp.int32),
            pltpu.VMEM((b_per_w, D), jnp.float32),
            pltpu.SemaphoreType.DMA,
        ],
    )
    def gather_rows(table_hbm, idx_hbm, out_hbm, idx_v, rows_v, sem):
        wid = lax.axis_index("s") * info.num_cores + lax.axis_index("c")
        base = wid * b_per_w
        pltpu.sync_copy(idx_hbm.at[pl.ds(base, b_per_w)], idx_v)
        pltpu.async_copy(table_hbm.at[idx_v], rows_v, sem).wait()
        pltpu.sync_copy(rows_v, out_hbm.at[pl.ds(base, b_per_w)])

    return gather_rows


def kernel(z, embedding):
    zbb, zn, ebb, en = _prep(z, embedding)
    idx = _encode_indices(zbb, zn, ebb, en)
    idx_flat = idx.reshape(B * T)
    gathered = _make_sc_gather()(embedding, idx_flat)
    return _transpose_btd(gathered.reshape(B, T, D))


# speedup vs baseline: 2.0511x; 1.0070x over previous
import functools

import jax
import jax.numpy as jnp
from jax import lax
from jax.experimental import pallas as pl
from jax.experimental.pallas import tpu as pltpu
from jax.experimental.pallas import tpu_sc as plsc

B, D, T = 16, 256, 576
K = 8192
BK = 8192
NKB = K // BK
G = 8
NH = 32
BKH = BK // NH
J = 8


EPB = K // B


def _prep_body(z_ref, emb_ref, zbb_ref, zn_ref, ebb_ref, en_ref):
    zb = z_ref[0]
    zbb_ref[0] = zb.astype(jnp.bfloat16)
    zn_ref[0] = jnp.sum(zb * zb, axis=0, keepdims=True)
    eb = emb_ref[...]
    ebb_ref[0] = (eb * -2.0).astype(jnp.bfloat16)
    en_ref[0] = jnp.sum(eb * eb, axis=1, keepdims=True)


def _prep(z, embedding):
    zbb, zn, ebb, en = pl.pallas_call(
        _prep_body,
        grid=(B,),
        in_specs=[
            pl.BlockSpec((1, D, T), lambda b: (b, 0, 0)),
            pl.BlockSpec((EPB, D), lambda b: (b, 0)),
        ],
        out_specs=[
            pl.BlockSpec((1, D, T), lambda b: (b, 0, 0)),
            pl.BlockSpec((1, 1, T), lambda b: (b, 0, 0)),
            pl.BlockSpec((1, EPB, D), lambda b: (b, 0, 0)),
            pl.BlockSpec((1, EPB, 1), lambda b: (b, 0, 0)),
        ],
        out_shape=[
            jax.ShapeDtypeStruct((B, D, T), jnp.bfloat16),
            jax.ShapeDtypeStruct((B, 1, T), jnp.float32),
            jax.ShapeDtypeStruct((B, EPB, D), jnp.bfloat16),
            jax.ShapeDtypeStruct((B, EPB, 1), jnp.float32),
        ],
    )(z, embedding)
    return (zbb, zn, ebb.reshape(NKB, BK, D), en.reshape(NKB, BK, 1))


def _argmin_body(zbb_ref, zn_ref, ebb_ref, en_ref, idx_ref, minval_ref):
    kblk = pl.program_id(1)
    zn = zn_ref[0]

    gph = BKH // G
    minvs = [jnp.full((G, T), jnp.inf, dtype=jnp.float32) for _ in range(J)]
    argis = [jnp.zeros((G, T), dtype=jnp.float32) for _ in range(J)]
    for h in range(NH):
        mm2h = lax.dot_general(
            ebb_ref[0, pl.ds(h * BKH, BKH), :], zbb_ref[0],
            (((1,), (0,)), ((), ())),
            preferred_element_type=jnp.float32)
        for gg in range(gph):
            g = h * gph + gg
            j = g % J
            ens = en_ref[0, pl.ds(g * G, G), :]
            mm2s = mm2h[gg * G:(gg + 1) * G, :]
            dd = (zn + ens) + mm2s
            win = dd < minvs[j]
            argis[j] = jnp.where(win, jnp.float32(g), argis[j])
            minvs[j] = jnp.where(win, dd, minvs[j])

    srow = lax.broadcasted_iota(jnp.int32, (G, T), 0).astype(jnp.float32)
    vidxs = [a * jnp.float32(G) + srow for a in argis]
    nacc = J
    while nacc > 1:
        nacc //= 2
        for j in range(nacc):
            va, vb = minvs[j], minvs[j + nacc]
            ia, ib = vidxs[j], vidxs[j + nacc]
            ta = (va < vb) | ((va == vb) & (ia < ib))
            minvs[j] = jnp.where(ta, va, vb)
            vidxs[j] = jnp.where(ta, ia, ib)
    minv, vidx = minvs[0], vidxs[0]
    h = G
    while h > 1:
        h //= 2
        va, vb = minv[:h], minv[h:]
        ia, ib = vidx[:h], vidx[h:]
        ta = (va < vb) | ((va == vb) & (ia < ib))
        minv = jnp.where(ta, va, vb)
        vidx = jnp.where(ta, ia, ib)
    lmin = minv
    larg = vidx.astype(jnp.int32) + kblk * BK

    prev = minval_ref[...]
    better = jnp.logical_or(lmin < prev, kblk == 0)
    minval_ref[...] = jnp.where(better, lmin, prev)
    idx_ref[0] = jnp.where(better, larg, idx_ref[0])


def _encode_indices(zbb, zn, ebb, en):
    return pl.pallas_call(
        _argmin_body,
        grid=(B, NKB),
        in_specs=[
            pl.BlockSpec((1, D, T), lambda b, k: (b, 0, 0)),
            pl.BlockSpec((1, 1, T), lambda b, k: (b, 0, 0)),
            pl.BlockSpec((1, BK, D), lambda b, k: (k, 0, 0)),
            pl.BlockSpec((1, BK, 1), lambda b, k: (k, 0, 0)),
        ],
        out_specs=pl.BlockSpec((1, 1, T), lambda b, k: (b, 0, 0)),
        out_shape=jax.ShapeDtypeStruct((B, 1, T), jnp.int32),
        scratch_shapes=[
            pltpu.VMEM((1, T), jnp.float32),
        ],
    )(zbb, zn, ebb, en)


TB = 4


def _transpose_body(in_ref, out_ref):
    for i in range(TB):
        out_ref[i] = in_ref[i].T


def _transpose_btd(x):
    return pl.pallas_call(
        _transpose_body,
        grid=(B // TB,),
        in_specs=[pl.BlockSpec((TB, T, D), lambda b: (b, 0, 0))],
        out_specs=pl.BlockSpec((TB, D, T), lambda b: (b, 0, 0)),
        out_shape=jax.ShapeDtypeStruct((B, D, T), jnp.float32),
    )(x)


def _make_sc_gather():
    info = plsc.get_sparse_core_info()
    nw = info.num_cores * info.num_subcores
    rows = B * T
    b_per_w = rows // nw
    mesh = plsc.VectorSubcoreMesh(core_axis_name="c", subcore_axis_name="s")

    half = b_per_w // 2

    @functools.partial(
        pl.kernel, mesh=mesh,
        out_type=jax.ShapeDtypeStruct((rows, D), jnp.float32),
        scratch_types=[
            pltpu.VMEM((b_per_w,), jnp.int32),
            pltpu.VMEM((2, half, D), jnp.float32),
            pltpu.SemaphoreType.DMA,
            pltpu.SemaphoreType.DMA,
        ],
    )
    def gather_rows(table_hbm, idx_hbm, out_hbm, idx_v, rows_v, sem0, sem1):
        wid = lax.axis_index("s") * info.num_cores + lax.axis_index("c")
        base = wid * b_per_w
        pltpu.sync_copy(idx_hbm.at[pl.ds(base, b_per_w)], idx_v)
        g0 = pltpu.async_copy(table_hbm.at[idx_v.at[pl.ds(0, half)]],
                              rows_v.at[0], sem0)
        g1 = pltpu.async_copy(table_hbm.at[idx_v.at[pl.ds(half, half)]],
                              rows_v.at[1], sem1)
        g0.wait()
        pltpu.sync_copy(rows_v.at[0], out_hbm.at[pl.ds(base, half)])
        g1.wait()
        pltpu.sync_copy(rows_v.at[1], out_hbm.at[pl.ds(base + half, half)])

    return gather_rows


def kernel(z, embedding):
    zbb, zn, ebb, en = _prep(z, embedding)
    idx = _encode_indices(zbb, zn, ebb, en)
    idx_flat = idx.reshape(B * T)
    gathered = _make_sc_gather()(embedding, idx_flat)
    return _transpose_btd(gathered.reshape(B, T, D))


# speedup vs baseline: 2.1241x; 1.0356x over previous
import functools

import jax
import jax.numpy as jnp
from jax import lax
from jax.experimental import pallas as pl
from jax.experimental.pallas import tpu as pltpu
from jax.experimental.pallas import tpu_sc as plsc

B, D, T = 16, 256, 576
K = 8192
BK = 8192
NKB = K // BK
G = 8
NH = 32
BKH = BK // NH
J = 8


EPB = K // B


def _prep_body(emb_ref, ebb_ref, en_ref):
    eb = emb_ref[...]
    ebb_ref[0] = (eb * -2.0).astype(jnp.bfloat16)
    en_ref[0] = jnp.sum(eb * eb, axis=1, keepdims=True)


def _prep(embedding):
    ebb, en = pl.pallas_call(
        _prep_body,
        grid=(B,),
        in_specs=[pl.BlockSpec((EPB, D), lambda b: (b, 0))],
        out_specs=[
            pl.BlockSpec((1, EPB, D), lambda b: (b, 0, 0)),
            pl.BlockSpec((1, EPB, 1), lambda b: (b, 0, 0)),
        ],
        out_shape=[
            jax.ShapeDtypeStruct((B, EPB, D), jnp.bfloat16),
            jax.ShapeDtypeStruct((B, EPB, 1), jnp.float32),
        ],
    )(embedding)
    return ebb.reshape(NKB, BK, D), en.reshape(NKB, BK, 1)


def _argmin_body(z_ref, ebb_ref, en_ref, idx_ref, minval_ref, zbb_ref):
    kblk = pl.program_id(1)
    zb = z_ref[0]
    zn = jnp.sum(zb * zb, axis=0, keepdims=True)
    zbb_ref[...] = zb.astype(jnp.bfloat16)

    gph = BKH // G
    minvs = [jnp.full((G, T), jnp.inf, dtype=jnp.float32) for _ in range(J)]
    argis = [jnp.zeros((G, T), dtype=jnp.float32) for _ in range(J)]
    for h in range(NH):
        mm2h = lax.dot_general(
            ebb_ref[0, pl.ds(h * BKH, BKH), :], zbb_ref[...],
            (((1,), (0,)), ((), ())),
            preferred_element_type=jnp.float32)
        for gg in range(gph):
            g = h * gph + gg
            j = g % J
            ens = en_ref[0, pl.ds(g * G, G), :]
            mm2s = mm2h[gg * G:(gg + 1) * G, :]
            dd = (zn + ens) + mm2s
            win = dd < minvs[j]
            argis[j] = jnp.where(win, jnp.float32(g), argis[j])
            minvs[j] = jnp.where(win, dd, minvs[j])

    srow = lax.broadcasted_iota(jnp.int32, (G, T), 0).astype(jnp.float32)
    vidxs = [a * jnp.float32(G) + srow for a in argis]
    nacc = J
    while nacc > 1:
        nacc //= 2
        for j in range(nacc):
            va, vb = minvs[j], minvs[j + nacc]
            ia, ib = vidxs[j], vidxs[j + nacc]
            ta = (va < vb) | ((va == vb) & (ia < ib))
            minvs[j] = jnp.where(ta, va, vb)
            vidxs[j] = jnp.where(ta, ia, ib)
    minv, vidx = minvs[0], vidxs[0]
    h = G
    while h > 1:
        h //= 2
        va, vb = minv[:h], minv[h:]
        ia, ib = vidx[:h], vidx[h:]
        ta = (va < vb) | ((va == vb) & (ia < ib))
        minv = jnp.where(ta, va, vb)
        vidx = jnp.where(ta, ia, ib)
    lmin = minv
    larg = vidx.astype(jnp.int32) + kblk * BK

    prev = minval_ref[...]
    better = jnp.logical_or(lmin < prev, kblk == 0)
    minval_ref[...] = jnp.where(better, lmin, prev)
    idx_ref[0] = jnp.where(better, larg, idx_ref[0])


def _encode_indices(z, ebb, en):
    return pl.pallas_call(
        _argmin_body,
        grid=(B, NKB),
        in_specs=[
            pl.BlockSpec((1, D, T), lambda b, k: (b, 0, 0)),
            pl.BlockSpec((1, BK, D), lambda b, k: (k, 0, 0)),
            pl.BlockSpec((1, BK, 1), lambda b, k: (k, 0, 0)),
        ],
        out_specs=pl.BlockSpec((1, 1, T), lambda b, k: (b, 0, 0)),
        out_shape=jax.ShapeDtypeStruct((B, 1, T), jnp.int32),
        scratch_shapes=[
            pltpu.VMEM((1, T), jnp.float32),
            pltpu.VMEM((D, T), jnp.bfloat16),
        ],
    )(z, ebb, en)


TB = 4


def _transpose_body(in_ref, out_ref):
    for i in range(TB):
        out_ref[i] = in_ref[i].T


def _transpose_btd(x):
    return pl.pallas_call(
        _transpose_body,
        grid=(B // TB,),
        in_specs=[pl.BlockSpec((TB, T, D), lambda b: (b, 0, 0))],
        out_specs=pl.BlockSpec((TB, D, T), lambda b: (b, 0, 0)),
        out_shape=jax.ShapeDtypeStruct((B, D, T), jnp.float32),
    )(x)


def _make_sc_gather():
    info = plsc.get_sparse_core_info()
    nw = info.num_cores * info.num_subcores
    rows = B * T
    b_per_w = rows // nw
    mesh = plsc.VectorSubcoreMesh(core_axis_name="c", subcore_axis_name="s")

    half = b_per_w // 2

    @functools.partial(
        pl.kernel, mesh=mesh,
        out_type=jax.ShapeDtypeStruct((rows, D), jnp.float32),
        scratch_types=[
            pltpu.VMEM((b_per_w,), jnp.int32),
            pltpu.VMEM((2, half, D), jnp.float32),
            pltpu.SemaphoreType.DMA,
            pltpu.SemaphoreType.DMA,
        ],
    )
    def gather_rows(table_hbm, idx_hbm, out_hbm, idx_v, rows_v, sem0, sem1):
        wid = lax.axis_index("s") * info.num_cores + lax.axis_index("c")
        base = wid * b_per_w
        pltpu.sync_copy(idx_hbm.at[pl.ds(base, b_per_w)], idx_v)
        g0 = pltpu.async_copy(table_hbm.at[idx_v.at[pl.ds(0, half)]],
                              rows_v.at[0], sem0)
        g1 = pltpu.async_copy(table_hbm.at[idx_v.at[pl.ds(half, half)]],
                              rows_v.at[1], sem1)
        g0.wait()
        pltpu.sync_copy(rows_v.at[0], out_hbm.at[pl.ds(base, half)])
        g1.wait()
        pltpu.sync_copy(rows_v.at[1], out_hbm.at[pl.ds(base + half, half)])

    return gather_rows


def kernel(z, embedding):
    ebb, en = _prep(embedding)
    idx = _encode_indices(z, ebb, en)
    idx_flat = idx.reshape(B * T)
    gathered = _make_sc_gather()(embedding, idx_flat)
    return _transpose_btd(gathered.reshape(B, T, D))
